# baseline (speedup 1.0000x reference)
import functools

import jax
import jax.numpy as jnp
from jax import lax
from jax.experimental import pallas as pl
from jax.experimental.pallas import tpu as pltpu
from jax.experimental.pallas import tpu_sc as plsc

D = 1024

NC, NS = 2, 16
NW = NC * NS

CHUNK = 16

TM = 512


def _embed_gather(idx_flat, tok_table):
    bt = idx_flat.shape[0]
    rows_per_w = bt // NW
    n_chunks = rows_per_w // CHUNK
    mesh = plsc.VectorSubcoreMesh(core_axis_name="c", subcore_axis_name="s")

    @functools.partial(
        pl.kernel,
        out_type=jax.ShapeDtypeStruct((bt, D), jnp.float32),
        mesh=mesh,
        scratch_types=[
            pltpu.VMEM((rows_per_w,), jnp.int32),
            pltpu.VMEM((CHUNK, D), jnp.float32),
            pltpu.VMEM((CHUNK, D), jnp.float32),
            pltpu.VMEM((CHUNK, D), jnp.float32),
            pltpu.VMEM((CHUNK, D), jnp.float32),
            pltpu.SemaphoreType.DMA,
            pltpu.SemaphoreType.DMA,
            pltpu.SemaphoreType.DMA,
            pltpu.SemaphoreType.DMA,
            pltpu.SemaphoreType.DMA,
            pltpu.SemaphoreType.DMA,
            pltpu.SemaphoreType.DMA,
            pltpu.SemaphoreType.DMA,
        ],
    )
    def k(idx_hbm, tok_hbm, x_hbm, idx_v, rows0, rows1, rows2, rows3,
          g0, g1, g2, g3, s0, s1, s2, s3):
        wid = lax.axis_index("s") * NC + lax.axis_index("c")
        base = wid * rows_per_w
        bufs = (rows0, rows1, rows2, rows3)
        gsems = (g0, g1, g2, g3)
        ssems = (s0, s1, s2, s3)
        nb = 4

        def gather(c):
            return pltpu.make_async_copy(
                tok_hbm.at[idx_v.at[pl.ds(c * CHUNK, CHUNK)]],
                bufs[c % nb], gsems[c % nb])

        def store(c):
            return pltpu.make_async_copy(
                bufs[c % nb], x_hbm.at[pl.ds(base + c * CHUNK, CHUNK)],
                ssems[c % nb])

        pltpu.sync_copy(idx_hbm.at[pl.ds(base, rows_per_w)], idx_v)
        gather(0).start()
        gather(1).start()
        gather(2).start()
        for c in range(n_chunks):
            if c + 3 < n_chunks:
                if c >= 1:
                    store(c - 1).wait()
                gather(c + 3).start()
            gather(c).wait()
            store(c).start()
        store(n_chunks - 4).wait()
        store(n_chunks - 3).wait()
        store(n_chunks - 2).wait()
        store(n_chunks - 1).wait()

    return k(idx_flat, tok_table)


def _mm_body(x_ref, pos_ref, w_ref, b_ref, o_ref):
    xs = (x_ref[...] + pos_ref[...]).astype(jnp.bfloat16)
    acc = lax.dot_general(
        xs, w_ref[...], (((1,), (1,)), ((), ())),
        preferred_element_type=jnp.float32,
    )
    o_ref[...] = acc + b_ref[...]


def _matmul(x, pos_table, w_bf16, b2):
    bt = x.shape[0]
    v = w_bf16.shape[0]
    t_len = pos_table.shape[0]
    t_tiles = t_len // TM
    b_tiles = bt // t_len
    return pl.pallas_call(
        _mm_body,
        grid=(bt // TM,),
        in_specs=[
            pl.BlockSpec((TM, D),
                         lambda i: ((i % b_tiles) * t_tiles + i // b_tiles, 0)),
            pl.BlockSpec((TM, D), lambda i: (i // b_tiles, 0)),
            pl.BlockSpec((v, D), lambda i: (0, 0)),
            pl.BlockSpec((1, v), lambda i: (0, 0)),
        ],
        out_specs=pl.BlockSpec(
            (TM, v), lambda i: ((i % b_tiles) * t_tiles + i // b_tiles, 0)),
        out_shape=jax.ShapeDtypeStruct((bt, v), jnp.float32),
    )(x, pos_table, w_bf16, b2)


def kernel(idx, tok_table, pos_table, W, b):
    B, T = idx.shape
    v = W.shape[0]
    idx_flat = idx.reshape(-1).astype(jnp.int32)
    w_bf16 = W.astype(jnp.bfloat16)
    x = _embed_gather(idx_flat, tok_table)
    logits = _matmul(x, pos_table, w_bf16, b.reshape(1, -1))
    return logits.reshape(B, T, v)


# speedup vs baseline: 1.0050x; 1.0050x over previous
import functools

import jax
import jax.numpy as jnp
from jax import lax
from jax.experimental import pallas as pl
from jax.experimental.pallas import tpu as pltpu
from jax.experimental.pallas import tpu_sc as plsc

D = 1024

NC, NS = 2, 16
NW = NC * NS

CHUNK = 32

TM = 512


def _embed_gather(idx_flat, tok_table):
    bt = idx_flat.shape[0]
    rows_per_w = bt // NW
    n_chunks = rows_per_w // CHUNK
    mesh = plsc.VectorSubcoreMesh(core_axis_name="c", subcore_axis_name="s")

    @functools.partial(
        pl.kernel,
        out_type=jax.ShapeDtypeStruct((bt, D), jnp.float32),
        mesh=mesh,
        scratch_types=[
            pltpu.VMEM((rows_per_w,), jnp.int32),
            pltpu.VMEM((CHUNK, D), jnp.float32),
            pltpu.VMEM((CHUNK, D), jnp.float32),
            pltpu.VMEM((CHUNK, D), jnp.float32),
            pltpu.SemaphoreType.DMA,
            pltpu.SemaphoreType.DMA,
            pltpu.SemaphoreType.DMA,
            pltpu.SemaphoreType.DMA,
            pltpu.SemaphoreType.DMA,
            pltpu.SemaphoreType.DMA,
        ],
    )
    def k(idx_hbm, tok_hbm, x_hbm, idx_v, rows0, rows1, rows2,
          g0, g1, g2, s0, s1, s2):
        wid = lax.axis_index("s") * NC + lax.axis_index("c")
        base = wid * rows_per_w
        bufs = (rows0, rows1, rows2)
        gsems = (g0, g1, g2)
        ssems = (s0, s1, s2)
        nb = 3

        def gather(c):
            return pltpu.make_async_copy(
                tok_hbm.at[idx_v.at[pl.ds(c * CHUNK, CHUNK)]],
                bufs[c % nb], gsems[c % nb])

        def store(c):
            return pltpu.make_async_copy(
                bufs[c % nb], x_hbm.at[pl.ds(base + c * CHUNK, CHUNK)],
                ssems[c % nb])

        pltpu.sync_copy(idx_hbm.at[pl.ds(base, rows_per_w)], idx_v)
        gather(0).start()
        gather(1).start()
        for c in range(n_chunks):
            if c + 2 < n_chunks:
                if c >= 1:
                    store(c - 1).wait()
                gather(c + 2).start()
            gather(c).wait()
            store(c).start()
        store(n_chunks - 3).wait()
        store(n_chunks - 2).wait()
        store(n_chunks - 1).wait()

    return k(idx_flat, tok_table)


def _mm_body(x_ref, pos_ref, w_ref, b_ref, o_ref):
    xs = (x_ref[...] + pos_ref[...]).astype(jnp.bfloat16)
    acc = lax.dot_general(
        xs, w_ref[...], (((1,), (1,)), ((), ())),
        preferred_element_type=jnp.float32,
    )
    o_ref[...] = acc + b_ref[...]


def _matmul(x, pos_table, w_bf16, b2):
    bt = x.shape[0]
    v = w_bf16.shape[0]
    t_len = pos_table.shape[0]
    t_tiles = t_len // TM
    b_tiles = bt // t_len
    return pl.pallas_call(
        _mm_body,
        grid=(bt // TM,),
        in_specs=[
            pl.BlockSpec((TM, D),
                         lambda i: ((i % b_tiles) * t_tiles + i // b_tiles, 0)),
            pl.BlockSpec((TM, D), lambda i: (i // b_tiles, 0)),
            pl.BlockSpec((v, D), lambda i: (0, 0)),
            pl.BlockSpec((1, v), lambda i: (0, 0)),
        ],
        out_specs=pl.BlockSpec(
            (TM, v), lambda i: ((i % b_tiles) * t_tiles + i // b_tiles, 0)),
        out_shape=jax.ShapeDtypeStruct((bt, v), jnp.float32),
    )(x, pos_table, w_bf16, b2)


def kernel(idx, tok_table, pos_table, W, b):
    B, T = idx.shape
    v = W.shape[0]
    idx_flat = idx.reshape(-1).astype(jnp.int32)
    w_bf16 = W.astype(jnp.bfloat16)
    x = _embed_gather(idx_flat, tok_table)
    logits = _matmul(x, pos_table, w_bf16, b.reshape(1, -1))
    return logits.reshape(B, T, v)
